# baseline (speedup 1.0000x reference)
import jax
import jax.numpy as jnp
from jax.experimental import pallas as pl
from functools import partial

B = 32
C_IN = 96
H = 224
W_SP = 224
HW = H * W_SP
OUT_CH = 96
GROUP = 2
HIDDEN = OUT_CH // GROUP
K_ZERO = 23

ROWS = B * C_IN
ROW_TILE = 64


def _pool_sum_kernel(x_ref, out_ref):
    out_ref[:, :] = jnp.sum(x_ref[:, :], axis=1, keepdims=True)


def _epilogue_kernel(sums_ref, wt_ref, b_ref, lasso_ref, mask_ref, lasso_out_ref):
    pooled = sums_ref[:, :] * (1.0 / HW)
    logits = jnp.dot(pooled, wt_ref[:, :],
                     preferred_element_type=jnp.float32) + b_ref[:, :]
    s = jax.nn.sigmoid(logits)
    lasso_out_ref[:, :] = lasso_ref[:, :] + jnp.sum(s) * (1.0 / B)
    s_i = s[:, :, None]
    s_j = s[:, None, :]
    i_idx = jax.lax.broadcasted_iota(jnp.int32, (B, HIDDEN, HIDDEN), 1)
    j_idx = jax.lax.broadcasted_iota(jnp.int32, (B, HIDDEN, HIDDEN), 2)
    lt = s_j < s_i
    eq_lo = (s_j == s_i) & (j_idx < i_idx)
    rank = jnp.sum((lt | eq_lo).astype(jnp.int32), axis=2)
    mask_ref[:, :] = ((rank >= K_ZERO) & (s > 0.0)).astype(jnp.int32)


def kernel(x, lasso_sum, W, b):
    x2 = x.reshape(ROWS, HW)
    sums = pl.pallas_call(
        _pool_sum_kernel,
        grid=(ROWS // ROW_TILE,),
        in_specs=[pl.BlockSpec((ROW_TILE, HW), lambda i: (i, 0))],
        out_specs=pl.BlockSpec((ROW_TILE, 1), lambda i: (i, 0)),
        out_shape=jax.ShapeDtypeStruct((ROWS, 1), jnp.float32),
    )(x2)

    pooled_sums = sums.reshape(B, C_IN)
    wt = W.T
    b2 = b.reshape(1, HIDDEN)
    lasso2 = lasso_sum.reshape(1, 1)

    mask, lasso_out = pl.pallas_call(
        _epilogue_kernel,
        in_specs=[
            pl.BlockSpec((B, C_IN), lambda: (0, 0)),
            pl.BlockSpec((C_IN, HIDDEN), lambda: (0, 0)),
            pl.BlockSpec((1, HIDDEN), lambda: (0, 0)),
            pl.BlockSpec((1, 1), lambda: (0, 0)),
        ],
        out_specs=[
            pl.BlockSpec((B, HIDDEN), lambda: (0, 0)),
            pl.BlockSpec((1, 1), lambda: (0, 0)),
        ],
        out_shape=[
            jax.ShapeDtypeStruct((B, HIDDEN), jnp.int32),
            jax.ShapeDtypeStruct((1, 1), jnp.float32),
        ],
    )(pooled_sums, wt, b2, lasso2)

    expanded = jnp.reshape(
        jnp.broadcast_to(mask[:, :, None], (B, HIDDEN, GROUP)), (B, OUT_CH)
    )
    return expanded, lasso_out.reshape(())


# speedup vs baseline: 3.9906x; 3.9906x over previous
import jax
import jax.numpy as jnp
from jax.experimental import pallas as pl
from functools import partial

B = 32
C_IN = 96
H = 224
W_SP = 224
HW = H * W_SP
OUT_CH = 96
GROUP = 2
HIDDEN = OUT_CH // GROUP
K_ZERO = 23

C_TILE = 2


def _pool_sum_kernel(x_ref, out_ref):
    out_ref[0] = jnp.sum(x_ref[...], axis=(2, 3))


def _epilogue_kernel(sums_ref, wt_ref, b_ref, lasso_ref, mask_ref, lasso_out_ref):
    pooled = sums_ref[:, :] * (1.0 / HW)
    logits = jnp.dot(pooled, wt_ref[:, :],
                     preferred_element_type=jnp.float32) + b_ref[:, :]
    s = jax.nn.sigmoid(logits)
    lasso_out_ref[:, :] = lasso_ref[:, :] + jnp.sum(s) * (1.0 / B)
    s_i = s[:, :, None]
    s_j = s[:, None, :]
    i_idx = jax.lax.broadcasted_iota(jnp.int32, (B, HIDDEN, HIDDEN), 1)
    j_idx = jax.lax.broadcasted_iota(jnp.int32, (B, HIDDEN, HIDDEN), 2)
    lt = s_j < s_i
    eq_lo = (s_j == s_i) & (j_idx < i_idx)
    rank = jnp.sum((lt | eq_lo).astype(jnp.int32), axis=2)
    mask_ref[:, :] = ((rank >= K_ZERO) & (s > 0.0)).astype(jnp.int32)


def kernel(x, lasso_sum, W, b):
    n_steps = C_IN // C_TILE
    sums = pl.pallas_call(
        _pool_sum_kernel,
        grid=(n_steps,),
        in_specs=[pl.BlockSpec((B, C_TILE, H, W_SP), lambda i: (0, i, 0, 0))],
        out_specs=pl.BlockSpec((1, B, C_TILE), lambda i: (i, 0, 0)),
        out_shape=jax.ShapeDtypeStruct((n_steps, B, C_TILE), jnp.float32),
    )(x)

    pooled_sums = sums.transpose(1, 0, 2).reshape(B, C_IN)
    wt = W.T
    b2 = b.reshape(1, HIDDEN)
    lasso2 = lasso_sum.reshape(1, 1)

    mask, lasso_out = pl.pallas_call(
        _epilogue_kernel,
        in_specs=[
            pl.BlockSpec((B, C_IN), lambda: (0, 0)),
            pl.BlockSpec((C_IN, HIDDEN), lambda: (0, 0)),
            pl.BlockSpec((1, HIDDEN), lambda: (0, 0)),
            pl.BlockSpec((1, 1), lambda: (0, 0)),
        ],
        out_specs=[
            pl.BlockSpec((B, HIDDEN), lambda: (0, 0)),
            pl.BlockSpec((1, 1), lambda: (0, 0)),
        ],
        out_shape=[
            jax.ShapeDtypeStruct((B, HIDDEN), jnp.int32),
            jax.ShapeDtypeStruct((1, 1), jnp.float32),
        ],
    )(pooled_sums, wt, b2, lasso2)

    expanded = jnp.reshape(
        jnp.broadcast_to(mask[:, :, None], (B, HIDDEN, GROUP)), (B, OUT_CH)
    )
    return expanded, lasso_out.reshape(())


# speedup vs baseline: 4.0171x; 1.0066x over previous
import jax
import jax.numpy as jnp
from jax.experimental import pallas as pl
from functools import partial

B = 32
C_IN = 96
H = 224
W_SP = 224
HW = H * W_SP
OUT_CH = 96
GROUP = 2
HIDDEN = OUT_CH // GROUP
K_ZERO = 23

def _pool_sum_kernel(x_ref, out_ref):
    out_ref[0] = jnp.sum(x_ref[...], axis=(2, 3))


def _epilogue_kernel(sums_ref, wt_ref, b_ref, lasso_ref, mask_ref, lasso_out_ref):
    pooled = sums_ref[:, :] * (1.0 / HW)
    logits = jnp.dot(pooled, wt_ref[:, :],
                     preferred_element_type=jnp.float32) + b_ref[:, :]
    s = jax.nn.sigmoid(logits)
    lasso_out_ref[:, :] = lasso_ref[:, :] + jnp.sum(s) * (1.0 / B)
    s_i = s[:, :, None]
    s_j = s[:, None, :]
    i_idx = jax.lax.broadcasted_iota(jnp.int32, (B, HIDDEN, HIDDEN), 1)
    j_idx = jax.lax.broadcasted_iota(jnp.int32, (B, HIDDEN, HIDDEN), 2)
    lt = s_j < s_i
    eq_lo = (s_j == s_i) & (j_idx < i_idx)
    rank = jnp.sum((lt | eq_lo).astype(jnp.int32), axis=2)
    mask_ref[:, :] = ((rank >= K_ZERO) & (s > 0.0)).astype(jnp.int32)


def kernel(x, lasso_sum, W, b):
    sums = pl.pallas_call(
        _pool_sum_kernel,
        grid=(B,),
        in_specs=[pl.BlockSpec((1, C_IN, H, W_SP), lambda i: (i, 0, 0, 0))],
        out_specs=pl.BlockSpec((1, 1, C_IN), lambda i: (i, 0, 0)),
        out_shape=jax.ShapeDtypeStruct((B, 1, C_IN), jnp.float32),
    )(x)

    pooled_sums = sums.reshape(B, C_IN)
    wt = W.T
    b2 = b.reshape(1, HIDDEN)
    lasso2 = lasso_sum.reshape(1, 1)

    mask, lasso_out = pl.pallas_call(
        _epilogue_kernel,
        in_specs=[
            pl.BlockSpec((B, C_IN), lambda: (0, 0)),
            pl.BlockSpec((C_IN, HIDDEN), lambda: (0, 0)),
            pl.BlockSpec((1, HIDDEN), lambda: (0, 0)),
            pl.BlockSpec((1, 1), lambda: (0, 0)),
        ],
        out_specs=[
            pl.BlockSpec((B, HIDDEN), lambda: (0, 0)),
            pl.BlockSpec((1, 1), lambda: (0, 0)),
        ],
        out_shape=[
            jax.ShapeDtypeStruct((B, HIDDEN), jnp.int32),
            jax.ShapeDtypeStruct((1, 1), jnp.float32),
        ],
    )(pooled_sums, wt, b2, lasso2)

    expanded = jnp.reshape(
        jnp.broadcast_to(mask[:, :, None], (B, HIDDEN, GROUP)), (B, OUT_CH)
    )
    return expanded, lasso_out.reshape(())


# speedup vs baseline: 4.0305x; 1.0033x over previous
import jax
import jax.numpy as jnp
from jax.experimental import pallas as pl
from jax.experimental.pallas import tpu as pltpu
from functools import partial

B = 32
C_IN = 96
H = 224
W_SP = 224
HW = H * W_SP
OUT_CH = 96
GROUP = 2
HIDDEN = OUT_CH // GROUP
K_ZERO = 23

def _pool_sum_kernel(x_ref, out_ref):
    out_ref[0] = jnp.sum(x_ref[...], axis=(2, 3))


def _epilogue_kernel(sums_ref, wt_ref, b_ref, lasso_ref, mask_ref, lasso_out_ref):
    pooled = sums_ref[:, :] * (1.0 / HW)
    logits = jnp.dot(pooled, wt_ref[:, :],
                     preferred_element_type=jnp.float32) + b_ref[:, :]
    s = jax.nn.sigmoid(logits)
    lasso_out_ref[:, :] = lasso_ref[:, :] + jnp.sum(s) * (1.0 / B)
    s_i = s[:, :, None]
    s_j = s[:, None, :]
    i_idx = jax.lax.broadcasted_iota(jnp.int32, (B, HIDDEN, HIDDEN), 1)
    j_idx = jax.lax.broadcasted_iota(jnp.int32, (B, HIDDEN, HIDDEN), 2)
    lt = s_j < s_i
    eq_lo = (s_j == s_i) & (j_idx < i_idx)
    rank = jnp.sum((lt | eq_lo).astype(jnp.int32), axis=2)
    mask_ref[:, :] = ((rank >= K_ZERO) & (s > 0.0)).astype(jnp.int32)


def kernel(x, lasso_sum, W, b):
    sums = pl.pallas_call(
        _pool_sum_kernel,
        grid=(B,),
        in_specs=[pl.BlockSpec((1, C_IN, H, W_SP), lambda i: (i, 0, 0, 0))],
        out_specs=pl.BlockSpec((1, 1, C_IN), lambda i: (i, 0, 0)),
        out_shape=jax.ShapeDtypeStruct((B, 1, C_IN), jnp.float32),
        compiler_params=pltpu.CompilerParams(
            dimension_semantics=("parallel",),
        ),
    )(x)

    pooled_sums = sums.reshape(B, C_IN)
    wt = W.T
    b2 = b.reshape(1, HIDDEN)
    lasso2 = lasso_sum.reshape(1, 1)

    mask, lasso_out = pl.pallas_call(
        _epilogue_kernel,
        in_specs=[
            pl.BlockSpec((B, C_IN), lambda: (0, 0)),
            pl.BlockSpec((C_IN, HIDDEN), lambda: (0, 0)),
            pl.BlockSpec((1, HIDDEN), lambda: (0, 0)),
            pl.BlockSpec((1, 1), lambda: (0, 0)),
        ],
        out_specs=[
            pl.BlockSpec((B, HIDDEN), lambda: (0, 0)),
            pl.BlockSpec((1, 1), lambda: (0, 0)),
        ],
        out_shape=[
            jax.ShapeDtypeStruct((B, HIDDEN), jnp.int32),
            jax.ShapeDtypeStruct((1, 1), jnp.float32),
        ],
    )(pooled_sums, wt, b2, lasso2)

    expanded = jnp.reshape(
        jnp.broadcast_to(mask[:, :, None], (B, HIDDEN, GROUP)), (B, OUT_CH)
    )
    return expanded, lasso_out.reshape(())
